# baseline (speedup 1.0000x reference)
import functools
import math

import jax
import jax.numpy as jnp
from jax import lax
from jax.experimental import pallas as pl
from jax.experimental.pallas import tpu as pltpu
from jax.experimental.pallas import tpu_sc as plsc

_NC = 2
_NS = 16
_NW = _NC * _NS
_L = 16

_CHUNK = 128
_NB = 4


@functools.lru_cache(maxsize=None)
def _build(N, V, D):
    b_per_w = N // _NW
    n_steps = b_per_w // _CHUNK
    scale = jnp.float32(math.sqrt(D))
    mesh = plsc.VectorSubcoreMesh(
        core_axis_name="c", subcore_axis_name="s",
        num_cores=_NC, num_subcores=_NS)

    @functools.partial(
        pl.kernel,
        mesh=mesh,
        out_type=jax.ShapeDtypeStruct((N, D), jnp.float32),
        scratch_types=[
            pltpu.VMEM((n_steps, _CHUNK), jnp.int32),
            pltpu.VMEM((_NB, _CHUNK, D), jnp.float32),
            pltpu.VMEM((_NB, _CHUNK, D), jnp.float32),
            pltpu.SemaphoreType.DMA,
            pltpu.SemaphoreType.DMA,
        ],
        compiler_params=pltpu.CompilerParams(use_tc_tiling_on_sc=False),
    )
    def gather_scale(idx_hbm, table_hbm, out_hbm, idx_v, in_v, out_v,
                     gsem, ssem):
        wid = lax.axis_index("s") * _NC + lax.axis_index("c")
        base = pl.multiple_of(wid * b_per_w, b_per_w)

        pltpu.sync_copy(
            idx_hbm.at[pl.ds(pl.multiple_of(base // _CHUNK, n_steps),
                             n_steps)],
            idx_v)

        def fire_gather(g, b):
            return pltpu.async_copy(
                table_hbm.at[idx_v.at[g]], in_v.at[b], gsem)

        for b in range(_NB):
            fire_gather(b, b)

        def step(g, carry):
            b = lax.rem(g, _NB)
            pltpu.make_async_copy(table_hbm.at[idx_v.at[g]], in_v.at[b],
                                  gsem).wait()
            @pl.when(g >= _NB)
            def _():
                pltpu.make_async_copy(
                    out_v.at[b],
                    out_hbm.at[pl.ds(base, _CHUNK)],
                    ssem).wait()

            def scale_body(i, c2):
                r0 = i * 4
                for dr in range(4):
                    for j in range(D // _L):
                        sl = (r0 + dr, pl.ds(j * _L, _L))
                        out_v[(b,) + sl] = in_v[(b,) + sl] * scale
                return c2
            lax.fori_loop(0, _CHUNK // 4, scale_body, 0, unroll=2)

            pltpu.async_copy(
                out_v.at[b],
                out_hbm.at[pl.ds(pl.multiple_of(base + g * _CHUNK, _CHUNK),
                                 _CHUNK)],
                ssem)
            @pl.when(g + _NB < n_steps)
            def _():
                pltpu.async_copy(
                    table_hbm.at[idx_v.at[g + _NB]], in_v.at[b], gsem)
            return carry

        lax.fori_loop(0, n_steps, step, 0)

        for _ in range(_NB):
            pltpu.make_async_copy(
                out_v.at[0], out_hbm.at[pl.ds(base, _CHUNK)], ssem).wait()

    return gather_scale


def kernel(tokens, weight):
    B, S = tokens.shape
    V, D = weight.shape
    N = B * S
    idx2d = tokens.astype(jnp.int32).reshape(N // _CHUNK, _CHUNK)
    out = _build(N, V, D)(idx2d, weight)
    return out.reshape(B, S, D)


# speedup vs baseline: 1.2636x; 1.2636x over previous
import functools
import math

import jax
import jax.numpy as jnp
from jax import lax
from jax.experimental import pallas as pl
from jax.experimental.pallas import tpu as pltpu
from jax.experimental.pallas import tpu_sc as plsc

_NC = 2
_NS = 16
_NW = _NC * _NS
_L = 16

_CHUNK = 128
_K = 2
_MACRO = _CHUNK * _K


@functools.lru_cache(maxsize=None)
def _build(N, V, D):
    b_per_w = N // _NW
    n_idx_rows = b_per_w // _CHUNK
    n_macro = b_per_w // _MACRO
    n_pairs = n_macro // 2
    scale = jnp.float32(math.sqrt(D))
    mesh = plsc.VectorSubcoreMesh(
        core_axis_name="c", subcore_axis_name="s",
        num_cores=_NC, num_subcores=_NS)

    @functools.partial(
        pl.kernel,
        mesh=mesh,
        out_type=jax.ShapeDtypeStruct((N, D), jnp.float32),
        scratch_types=[
            pltpu.VMEM((n_idx_rows, _CHUNK), jnp.int32),
            pltpu.VMEM((_MACRO, D), jnp.float32),
            pltpu.VMEM((_MACRO, D), jnp.float32),
            pltpu.VMEM((_MACRO, D), jnp.float32),
            pltpu.VMEM((_MACRO, D), jnp.float32),
            pltpu.SemaphoreType.DMA,
            pltpu.SemaphoreType.DMA,
        ],
        compiler_params=pltpu.CompilerParams(use_tc_tiling_on_sc=False),
    )
    def gather_scale(idx_hbm, table_hbm, out_hbm, idx_v,
                     in0, in1, out0, out1, gsem, ssem):
        wid = lax.axis_index("s") * _NC + lax.axis_index("c")
        base = pl.multiple_of(wid * b_per_w, b_per_w)

        pltpu.sync_copy(
            idx_hbm.at[pl.ds(pl.multiple_of(base // _CHUNK, n_idx_rows),
                             n_idx_rows)],
            idx_v)

        def fire_gathers(m, buf):
            for j in range(_K):
                pltpu.async_copy(
                    table_hbm.at[idx_v.at[m * _K + j]],
                    buf.at[pl.ds(j * _CHUNK, _CHUNK)],
                    gsem)

        def drain_gathers(buf):
            for j in range(_K):
                pltpu.make_async_copy(
                    table_hbm.at[idx_v.at[0]],
                    buf.at[pl.ds(j * _CHUNK, _CHUNK)],
                    gsem).wait()

        def scale_buf(src, dst):
            def body(i, c):
                r0 = i * 4
                for dr in range(4):
                    for j in range(D // _L):
                        sl = (r0 + dr, pl.ds(j * _L, _L))
                        dst[sl] = src[sl] * scale
                return c
            lax.fori_loop(0, _MACRO // 4, body, 0)

        def out_slice(m):
            return out_hbm.at[
                pl.ds(pl.multiple_of(base + m * _MACRO, _MACRO), _MACRO)]

        def drain_store(buf):
            pltpu.make_async_copy(buf, out_slice(0), ssem).wait()

        def step(m, inb, outb):
            drain_gathers(inb)
            @pl.when(m >= 2)
            def _():
                drain_store(outb)
            scale_buf(inb, outb)
            pltpu.async_copy(outb, out_slice(m), ssem)
            @pl.when(m + 2 < n_macro)
            def _():
                fire_gathers(m + 2, inb)

        fire_gathers(0, in0)
        fire_gathers(1, in1)

        def pair(i, carry):
            m0 = i * 2
            step(m0, in0, out0)
            step(m0 + 1, in1, out1)
            return carry

        lax.fori_loop(0, n_pairs, pair, 0)

        drain_store(out0)
        drain_store(out1)

    return gather_scale


def kernel(tokens, weight):
    B, S = tokens.shape
    V, D = weight.shape
    N = B * S
    idx2d = tokens.astype(jnp.int32).reshape(N // _CHUNK, _CHUNK)
    out = _build(N, V, D)(idx2d, weight)
    return out.reshape(B, S, D)


# speedup vs baseline: 1.4062x; 1.1129x over previous
import functools
import math

import jax
import jax.numpy as jnp
from jax import lax
from jax.experimental import pallas as pl
from jax.experimental.pallas import tpu as pltpu
from jax.experimental.pallas import tpu_sc as plsc

_NC = 2
_NS = 16
_NW = _NC * _NS
_L = 16

_CHUNK = 128


@functools.lru_cache(maxsize=None)
def _build(B, S, V, D):
    n_tb = B // _CHUNK
    n_s8 = S // 8
    scale = jnp.float32(math.sqrt(D))
    mesh = plsc.VectorSubcoreMesh(
        core_axis_name="c", subcore_axis_name="s",
        num_cores=_NC, num_subcores=_NS)

    @functools.partial(
        pl.kernel,
        mesh=mesh,
        out_type=jax.ShapeDtypeStruct((S, D // 8, n_tb, 8, _CHUNK),
                                      jnp.float32),
        scratch_types=[
            pltpu.VMEM((n_s8, 1, 8, _CHUNK), jnp.int32),
            pltpu.VMEM((_CHUNK, D), jnp.float32),
            pltpu.VMEM((_CHUNK, D), jnp.float32),
            pltpu.VMEM((D // 8, 1, 8, _CHUNK), jnp.float32),
            pltpu.VMEM((D // 8, 1, 8, _CHUNK), jnp.float32),
            pltpu.SemaphoreType.DMA,
            pltpu.SemaphoreType.DMA,
        ],
        compiler_params=pltpu.CompilerParams(use_tc_tiling_on_sc=False,
                                             needs_layout_passes=False),
    )
    def gather_tr(tok_hbm, table_hbm, out_hbm, idx_v,
                  in0, in1, tr0, tr1, gsem, ssem):
        w = lax.axis_index("s") * _NC + lax.axis_index("c")

        pltpu.sync_copy(tok_hbm.at[:, pl.ds(w, 1)], idx_v)

        lanes = lax.iota(jnp.int32, _L)
        rd_row = []
        rd_col = []
        wr_e = []
        for k in range(_L):
            d = lax.rem(lanes + k, _L)
            rd_row.append(lanes)
            rd_col.append(d)
            wr_e.append(d)

        def fire_gather(s, buf):
            return pltpu.async_copy(
                table_hbm.at[idx_v.at[s // 8, 0, s % 8]], buf, gsem)

        def drain_gather(buf):
            pltpu.make_async_copy(
                table_hbm.at[idx_v.at[0, 0, 0]], buf, gsem).wait()

        def transpose_scale(src, dst):
            def body(i, c):
                cb = i // (D // _L)
                eb = lax.rem(i, D // _L)
                c0 = cb * _L
                e0 = eb * _L
                for k in range(_L):
                    v = plsc.load_gather(src, [rd_row[k] + c0,
                                               rd_col[k] + e0])
                    e = wr_e[k] + e0
                    plsc.store_scatter(
                        dst,
                        [lax.div(e, 8), jnp.zeros_like(e),
                         lax.rem(e, 8), rd_row[k] + c0],
                        v * scale)
                return c
            lax.fori_loop(0, (_CHUNK // _L) * (D // _L), body, 0)

        def out_slice(s):
            return out_hbm.at[s, pl.ds(0, D // 8), pl.ds(w, 1)]

        def drain_store(buf):
            pltpu.make_async_copy(buf, out_slice(0), ssem).wait()

        fire_gather(0, in0)
        fire_gather(1, in1)

        def step(s, inb, trb):
            drain_gather(inb)
            @pl.when(s >= 2)
            def _():
                drain_store(trb)
            transpose_scale(inb, trb)
            pltpu.async_copy(trb, out_slice(s), ssem)
            @pl.when(s + 2 < S)
            def _():
                fire_gather(s + 2, inb)

        def pair(i, carry):
            s0 = i * 2
            step(s0, in0, tr0)
            step(s0 + 1, in1, tr1)
            return carry

        lax.fori_loop(0, S // 2, pair, 0)

        drain_store(tr0)
        drain_store(tr1)

    return gather_tr


def kernel(tokens, weight):
    B, S = tokens.shape
    V, D = weight.shape
    tok4d = (tokens.astype(jnp.int32).T
             .reshape(S // 8, 8, B // _CHUNK, _CHUNK)
             .transpose(0, 2, 1, 3))
    out5d = _build(B, S, V, D)(tok4d, weight)
    return out5d.transpose(2, 4, 0, 1, 3).reshape(B, S, D)
